# Initial kernel scaffold; baseline (speedup 1.0000x reference)
import jax
import jax.numpy as jnp
from jax.experimental import pallas as pl


def kernel(in_repr, user_idxs, user_to_delta_set, deltas):
    raise NotImplementedError("write your pallas kernel here")


# speedup vs baseline: 3.6212x; 3.6212x over previous
import functools

import jax
import jax.numpy as jnp
from jax import lax
from jax.experimental import pallas as pl
from jax.experimental.pallas import tpu as pltpu
from jax.experimental.pallas import tpu_sc as plsc

BATCH = 16384
D = 128
NSETS = 100
NUSERS = 100
NC = 2
NS = 16
NW = NC * NS
RPW = BATCH // NW
L = 16


def _body(in_hbm, uidx_hbm, u2d_hbm, deltas_hbm, out_hbm,
          idx_v, dsidx_v, u2d_v, delta_v, buf_v):
    c = lax.axis_index("c")
    s = lax.axis_index("s")
    wid = s * NC + c
    base = wid * RPW

    pltpu.sync_copy(uidx_hbm.at[pl.ds(base, RPW)], idx_v)
    pltpu.sync_copy(u2d_hbm, u2d_v)
    pltpu.sync_copy(deltas_hbm, delta_v)
    pltpu.sync_copy(in_hbm.at[pl.ds(base, RPW)], buf_v)

    def compose(i, carry):
        v = idx_v[pl.ds(i * L, L)]
        dsidx_v[pl.ds(i * L, L)] = plsc.load_gather(u2d_v, [v])
        return carry

    lax.fori_loop(0, RPW // L, compose, 0)

    def row(r, carry):
        di = dsidx_v[pl.ds(r, L)][0]
        for j in range(D // L):
            sl = pl.ds(j * L, L)
            buf_v[r, sl] = buf_v[r, sl] + delta_v[di, sl]
        return carry

    lax.fori_loop(0, RPW, row, 0)

    pltpu.sync_copy(buf_v, out_hbm.at[pl.ds(base, RPW)])


def kernel(in_repr, user_idxs, user_to_delta_set, deltas):
    mesh = plsc.VectorSubcoreMesh(core_axis_name="c", subcore_axis_name="s")
    k = pl.kernel(
        _body,
        mesh=mesh,
        out_type=jax.ShapeDtypeStruct((BATCH, D), jnp.float32),
        scratch_types=[
            pltpu.VMEM((RPW,), jnp.int32),
            pltpu.VMEM((RPW + L,), jnp.int32),
            pltpu.VMEM((NUSERS,), jnp.int32),
            pltpu.VMEM((NSETS, D), jnp.float32),
            pltpu.VMEM((RPW, D), jnp.float32),
        ],
        compiler_params=pltpu.CompilerParams(needs_layout_passes=False),
    )
    return k(in_repr, user_idxs, user_to_delta_set, deltas)


# speedup vs baseline: 6.2144x; 1.7161x over previous
import jax
import jax.numpy as jnp
from jax import lax
from jax.experimental import pallas as pl
from jax.experimental.pallas import tpu as pltpu
from jax.experimental.pallas import tpu_sc as plsc

BATCH = 16384
D = 128
NSETS = 100
NUSERS = 100
NC = 2
NS = 16
NW = NC * NS
RPW = BATCH // NW
L = 16
CH = 128
NCH = RPW // CH
G = 8


def _body(in_hbm, uidx_hbm, u2d_hbm, deltas_hbm, out_hbm,
          idx_v, dsidx_v, u2d_v, delta_v, bufs, in_sems, out_sems,
          idx_sem, u2d_sem, del_sem):
    c = lax.axis_index("c")
    s = lax.axis_index("s")
    wid = s * NC + c
    base = wid * RPW

    cp_idx = pltpu.async_copy(uidx_hbm.at[pl.ds(base, RPW)], idx_v, idx_sem)
    cp_u2d = pltpu.async_copy(u2d_hbm, u2d_v, u2d_sem)
    cp_del = pltpu.async_copy(deltas_hbm, delta_v, del_sem)
    cp_in = [
        pltpu.async_copy(in_hbm.at[pl.ds(base + k * CH, CH)], bufs[k],
                         in_sems[k])
        for k in range(NCH)
    ]

    cp_idx.wait()
    cp_u2d.wait()

    def compose(i, carry):
        v = idx_v[pl.ds(i * L, L)]
        dsidx_v[pl.ds(i * L, L)] = plsc.load_gather(u2d_v, [v])
        return carry

    lax.fori_loop(0, RPW // L, compose, 0)

    cp_del.wait()

    iota = lax.iota(jnp.int32, L)
    cols = [iota + (j * L) for j in range(D // L)]

    for k in range(NCH):
        cp_in[k].wait()
        buf = bufs[k]

        def group(g, carry, buf=buf, k=k):
            r0 = k * CH + g * G
            dvecs = [
                plsc.load_gather(dsidx_v,
                                 [jnp.full((L,), r0 + rr, jnp.int32)])
                for rr in range(G)
            ]
            for rr in range(G):
                rc = g * G + rr
                djs = [plsc.load_gather(delta_v, [dvecs[rr], cols[j]])
                       for j in range(D // L)]
                ins = [buf[rc, pl.ds(j * L, L)] for j in range(D // L)]
                for j in range(D // L):
                    buf[rc, pl.ds(j * L, L)] = ins[j] + djs[j]
            return carry

        lax.fori_loop(0, CH // G, group, 0)
        pltpu.async_copy(buf, out_hbm.at[pl.ds(base + k * CH, CH)],
                         out_sems[k])

    for k in range(NCH):
        pltpu.make_async_copy(bufs[k], out_hbm.at[pl.ds(base + k * CH, CH)],
                              out_sems[k]).wait()


def kernel(in_repr, user_idxs, user_to_delta_set, deltas):
    mesh = plsc.VectorSubcoreMesh(core_axis_name="c", subcore_axis_name="s")
    k = pl.kernel(
        _body,
        mesh=mesh,
        out_type=jax.ShapeDtypeStruct((BATCH, D), jnp.float32),
        scratch_types=[
            pltpu.VMEM((RPW,), jnp.int32),
            pltpu.VMEM((RPW,), jnp.int32),
            pltpu.VMEM((NUSERS,), jnp.int32),
            pltpu.VMEM((NSETS, D), jnp.float32),
            [pltpu.VMEM((CH, D), jnp.float32) for _ in range(NCH)],
            [pltpu.SemaphoreType.DMA for _ in range(NCH)],
            [pltpu.SemaphoreType.DMA for _ in range(NCH)],
            pltpu.SemaphoreType.DMA,
            pltpu.SemaphoreType.DMA,
            pltpu.SemaphoreType.DMA,
        ],
        compiler_params=pltpu.CompilerParams(needs_layout_passes=False),
    )
    return k(in_repr, user_idxs, user_to_delta_set, deltas)


# speedup vs baseline: 6.2350x; 1.0033x over previous
import jax
import jax.numpy as jnp
from jax import lax
from jax.experimental import pallas as pl
from jax.experimental.pallas import tpu as pltpu
from jax.experimental.pallas import tpu_sc as plsc

BATCH = 16384
D = 128
NSETS = 100
NUSERS = 100
NC = 2
NS = 16
NW = NC * NS
RPW = BATCH // NW
L = 16
CH = 256
NCH = RPW // CH
G = 8


def _body(in_hbm, uidx_hbm, u2d_hbm, deltas_hbm, out_hbm,
          idx_v, dsidx_v, u2d_v, delta_v, bufs, in_sems, out_sems,
          idx_sem, u2d_sem, del_sem):
    c = lax.axis_index("c")
    s = lax.axis_index("s")
    wid = s * NC + c
    base = wid * RPW

    cp_idx = pltpu.async_copy(uidx_hbm.at[pl.ds(base, RPW)], idx_v, idx_sem)
    cp_u2d = pltpu.async_copy(u2d_hbm, u2d_v, u2d_sem)
    cp_del = pltpu.async_copy(deltas_hbm, delta_v, del_sem)
    cp_in = [
        pltpu.async_copy(in_hbm.at[pl.ds(base + k * CH, CH)], bufs[k],
                         in_sems[k])
        for k in range(NCH)
    ]

    cp_idx.wait()
    cp_u2d.wait()

    def compose(i, carry):
        v = idx_v[pl.ds(i * L, L)]
        dsidx_v[pl.ds(i * L, L)] = plsc.load_gather(u2d_v, [v])
        return carry

    lax.fori_loop(0, RPW // L, compose, 0)

    cp_del.wait()

    iota = lax.iota(jnp.int32, L)
    cols = [iota + (j * L) for j in range(D // L)]

    for k in range(NCH):
        cp_in[k].wait()
        buf = bufs[k]

        def group(g, carry, buf=buf, k=k):
            r0 = k * CH + g * G
            dvecs = [
                plsc.load_gather(dsidx_v,
                                 [jnp.full((L,), r0 + rr, jnp.int32)])
                for rr in range(G)
            ]
            for rr in range(G):
                rc = g * G + rr
                djs = [plsc.load_gather(delta_v, [dvecs[rr], cols[j]])
                       for j in range(D // L)]
                ins = [buf[rc, pl.ds(j * L, L)] for j in range(D // L)]
                for j in range(D // L):
                    buf[rc, pl.ds(j * L, L)] = ins[j] + djs[j]
            return carry

        lax.fori_loop(0, CH // G, group, 0)
        pltpu.async_copy(buf, out_hbm.at[pl.ds(base + k * CH, CH)],
                         out_sems[k])

    for k in range(NCH):
        pltpu.make_async_copy(bufs[k], out_hbm.at[pl.ds(base + k * CH, CH)],
                              out_sems[k]).wait()


def kernel(in_repr, user_idxs, user_to_delta_set, deltas):
    mesh = plsc.VectorSubcoreMesh(core_axis_name="c", subcore_axis_name="s")
    k = pl.kernel(
        _body,
        mesh=mesh,
        out_type=jax.ShapeDtypeStruct((BATCH, D), jnp.float32),
        scratch_types=[
            pltpu.VMEM((RPW,), jnp.int32),
            pltpu.VMEM((RPW,), jnp.int32),
            pltpu.VMEM((NUSERS,), jnp.int32),
            pltpu.VMEM((NSETS, D), jnp.float32),
            [pltpu.VMEM((CH, D), jnp.float32) for _ in range(NCH)],
            [pltpu.SemaphoreType.DMA for _ in range(NCH)],
            [pltpu.SemaphoreType.DMA for _ in range(NCH)],
            pltpu.SemaphoreType.DMA,
            pltpu.SemaphoreType.DMA,
            pltpu.SemaphoreType.DMA,
        ],
        compiler_params=pltpu.CompilerParams(needs_layout_passes=False),
    )
    return k(in_repr, user_idxs, user_to_delta_set, deltas)
